# Initial kernel scaffold; baseline (speedup 1.0000x reference)
import jax
import jax.numpy as jnp
from jax.experimental import pallas as pl


def kernel(atom_inputs, element_embed, degree_embed, ring_embed, charge_embed, aromatic_embed, hybrid_embed, hydrogen_embed, func_embeds, h_don_embed, h_acc_embed, ringsize_embed, aroma_num_embed, fused_if_embed, func_reduce_w, func_reduce_b, bond_env_w, bond_env_b):
    raise NotImplementedError("write your pallas kernel here")


# speedup vs baseline: 25.6066x; 25.6066x over previous
import jax
import jax.numpy as jnp
import numpy as np
from jax.experimental import pallas as pl

_N = 100000
_BLOCK = 1000
_IN_W = 78
_OUT_W = 64
_NPOS = 160
_NIDX = 32

_ELEMENTS = (5, 6, 7, 8, 14, 15, 16)
_RINGSIZE_VALS = (0, 3, 4, 5, 6, 7, 8)

_TABLE_SIZES = (
    (0, 7), (1, 7), (5, 2), (2, 8), (4, 2), (3, 6), (6, 5),
    *(((7 + j), 2) for j in range(18)),
    (25, 2), (26, 2), (27, 7), (28, 5), (29, 8),
)


def _table_positions():
    starts = {}
    p = 0
    for col, size in _TABLE_SIZES:
        starts[col] = (p, size)
        p += size
    assert p == 97
    return starts


def _build_constants(element_embed, degree_embed, ring_embed, charge_embed,
                     aromatic_embed, hybrid_embed, hydrogen_embed, func_embeds,
                     h_don_embed, h_acc_embed, ringsize_embed, aroma_num_embed,
                     fused_if_embed, func_reduce_w, func_reduce_b,
                     bond_env_w, bond_env_b):
    starts = _table_positions()

    sel_np = np.zeros((_NIDX, _NPOS), dtype=np.float32)
    rvals_np = np.full((1, _NPOS), -1.0, dtype=np.float32)
    for col, size in _TABLE_SIZES:
        p0, _ = starts[col]
        sel_np[col, p0:p0 + size] = 1.0
        rvals_np[0, p0:p0 + size] = np.arange(size, dtype=np.float32)
    sel = jnp.asarray(sel_np)
    rvals = jnp.asarray(rvals_np)

    w_full = jnp.zeros((_NPOS, _OUT_W), dtype=jnp.float32)

    def place(w_full, col, table, out0):
        p0, size = starts[col]
        d = table.shape[1]
        return w_full.at[p0:p0 + size, out0:out0 + d].set(table)

    w_full = place(w_full, 0, element_embed, 0)
    w_full = place(w_full, 1, degree_embed, 4)
    w_full = place(w_full, 5, ring_embed, 8)
    w_full = place(w_full, 2, charge_embed, 12)
    w_full = place(w_full, 4, aromatic_embed, 16)
    w_full = place(w_full, 3, hybrid_embed, 20)
    w_full = place(w_full, 6, hydrogen_embed, 24)
    for j in range(18):
        fused_j = func_embeds[j] @ func_reduce_w[2 * j:2 * j + 2, :]
        w_full = place(w_full, 7 + j, fused_j, 28)
    w_full = place(w_full, 25, h_don_embed, 32)
    w_full = place(w_full, 26, h_acc_embed, 34)
    w_full = place(w_full, 27, ringsize_embed, 36)
    w_full = place(w_full, 28, aroma_num_embed, 40)
    w_full = place(w_full, 29, fused_if_embed, 44)

    w_a = jnp.zeros((_IN_W, _OUT_W), dtype=jnp.float32)
    w_a = w_a.at[30:78, 48:64].set(bond_env_w)

    bias = jnp.zeros((1, _OUT_W), dtype=jnp.float32)
    bias = bias.at[0, 28:32].set(func_reduce_b)
    bias = bias.at[0, 48:64].set(bond_env_b)

    return sel, rvals, w_full, w_a, bias


def _body(a_ref, hi_ref, sel_ref, rvals_ref, wfull_ref, wa_ref, bias_ref,
          o_ref):
    a = a_ref[...]
    cols = a[:, :30].astype(jnp.int32)

    z = cols[:, 0:1]
    idx0 = jnp.zeros_like(z)
    for i, zv in enumerate(_ELEMENTS):
        if i:
            idx0 = jnp.where(z == zv, i, idx0)

    cl = jnp.clip(cols, 0, hi_ref[:, :30])

    ring = jnp.clip(cols[:, 5:6] + 1, 0, 1)

    c27 = cols[:, 27:28]
    m27 = jnp.full_like(c27, 6)
    for i, v in enumerate(_RINGSIZE_VALS):
        if i != 6:
            m27 = jnp.where(c27 == v, i, m27)

    idx_all = jnp.concatenate(
        [idx0, cl[:, 1:5], ring, cl[:, 6:27], m27, cl[:, 28:30],
         jnp.zeros((a.shape[0], 2), dtype=jnp.int32)], axis=1)
    idx_f = idx_all.astype(jnp.float32)

    bcast = jax.lax.dot(idx_f, sel_ref[...],
                        preferred_element_type=jnp.float32)
    onehot = (bcast == rvals_ref[...]).astype(jnp.float32)

    out = jax.lax.dot(onehot, wfull_ref[...],
                      preferred_element_type=jnp.float32)
    out = out + jax.lax.dot(a, wa_ref[...],
                            preferred_element_type=jnp.float32)
    o_ref[...] = out + bias_ref[...]


def kernel(atom_inputs, element_embed, degree_embed, ring_embed, charge_embed,
           aromatic_embed, hybrid_embed, hydrogen_embed, func_embeds,
           h_don_embed, h_acc_embed, ringsize_embed, aroma_num_embed,
           fused_if_embed, func_reduce_w, func_reduce_b, bond_env_w,
           bond_env_b):
    sel, rvals, w_full, w_a, bias = _build_constants(
        element_embed, degree_embed, ring_embed, charge_embed, aromatic_embed,
        hybrid_embed, hydrogen_embed, func_embeds, h_don_embed, h_acc_embed,
        ringsize_embed, aroma_num_embed, fused_if_embed, func_reduce_w,
        func_reduce_b, bond_env_w, bond_env_b)

    hi = jnp.array([[0, 6, 7, 5, 1, 0, 4] + [1] * 18 + [1, 1, 0, 4, 7, 0, 0]],
                   dtype=jnp.int32)

    n = atom_inputs.shape[0]
    assert n % _BLOCK == 0
    grid = (n // _BLOCK,)

    full = lambda shape: pl.BlockSpec(shape, lambda i: (0, 0))
    return pl.pallas_call(
        _body,
        grid=grid,
        in_specs=[
            pl.BlockSpec((_BLOCK, _IN_W), lambda i: (i, 0)),
            full((1, _NIDX)),
            full((_NIDX, _NPOS)),
            full((1, _NPOS)),
            full((_NPOS, _OUT_W)),
            full((_IN_W, _OUT_W)),
            full((1, _OUT_W)),
        ],
        out_specs=pl.BlockSpec((_BLOCK, _OUT_W), lambda i: (i, 0)),
        out_shape=jax.ShapeDtypeStruct((n, _OUT_W), jnp.float32),
    )(atom_inputs, hi, sel, rvals, w_full, w_a, bias)


# speedup vs baseline: 27.5988x; 1.0778x over previous
import jax
import jax.numpy as jnp
import numpy as np
from jax.experimental import pallas as pl

_N = 100000
_BLOCK = 5000
_IN_W = 78
_OUT_W = 64
_NPOS = 160
_NIDX = 32

_ELEMENTS = (5, 6, 7, 8, 14, 15, 16)
_RINGSIZE_VALS = (0, 3, 4, 5, 6, 7, 8)

_TABLE_SIZES = (
    (0, 7), (1, 7), (5, 2), (2, 8), (4, 2), (3, 6), (6, 5),
    *(((7 + j), 2) for j in range(18)),
    (25, 2), (26, 2), (27, 7), (28, 5), (29, 8),
)


def _table_positions():
    starts = {}
    p = 0
    for col, size in _TABLE_SIZES:
        starts[col] = (p, size)
        p += size
    assert p == 97
    return starts


def _build_constants(element_embed, degree_embed, ring_embed, charge_embed,
                     aromatic_embed, hybrid_embed, hydrogen_embed, func_embeds,
                     h_don_embed, h_acc_embed, ringsize_embed, aroma_num_embed,
                     fused_if_embed, func_reduce_w, func_reduce_b,
                     bond_env_w, bond_env_b):
    starts = _table_positions()

    sel_np = np.zeros((_NIDX, _NPOS), dtype=np.float32)
    rvals_np = np.full((1, _NPOS), -1.0, dtype=np.float32)
    for col, size in _TABLE_SIZES:
        p0, _ = starts[col]
        sel_np[col, p0:p0 + size] = 1.0
        rvals_np[0, p0:p0 + size] = np.arange(size, dtype=np.float32)
    sel = jnp.asarray(sel_np)
    rvals = jnp.asarray(rvals_np)

    w_full = jnp.zeros((_NPOS, _OUT_W), dtype=jnp.float32)

    def place(w_full, col, table, out0):
        p0, size = starts[col]
        d = table.shape[1]
        return w_full.at[p0:p0 + size, out0:out0 + d].set(table)

    w_full = place(w_full, 0, element_embed, 0)
    w_full = place(w_full, 1, degree_embed, 4)
    w_full = place(w_full, 5, ring_embed, 8)
    w_full = place(w_full, 2, charge_embed, 12)
    w_full = place(w_full, 4, aromatic_embed, 16)
    w_full = place(w_full, 3, hybrid_embed, 20)
    w_full = place(w_full, 6, hydrogen_embed, 24)
    for j in range(18):
        fused_j = func_embeds[j] @ func_reduce_w[2 * j:2 * j + 2, :]
        w_full = place(w_full, 7 + j, fused_j, 28)
    w_full = place(w_full, 25, h_don_embed, 32)
    w_full = place(w_full, 26, h_acc_embed, 34)
    w_full = place(w_full, 27, ringsize_embed, 36)
    w_full = place(w_full, 28, aroma_num_embed, 40)
    w_full = place(w_full, 29, fused_if_embed, 44)

    w_a = jnp.zeros((_IN_W, _OUT_W), dtype=jnp.float32)
    w_a = w_a.at[30:78, 48:64].set(bond_env_w)

    bias = jnp.zeros((1, _OUT_W), dtype=jnp.float32)
    bias = bias.at[0, 28:32].set(func_reduce_b)
    bias = bias.at[0, 48:64].set(bond_env_b)

    return sel, rvals, w_full, w_a, bias


def _body(a_ref, hi_ref, sel_ref, rvals_ref, wfull_ref, wa_ref, bias_ref,
          o_ref):
    a = a_ref[...]
    cols = a[:, :30].astype(jnp.int32)

    z = cols[:, 0:1]
    idx0 = jnp.zeros_like(z)
    for i, zv in enumerate(_ELEMENTS):
        if i:
            idx0 = jnp.where(z == zv, i, idx0)

    cl = jnp.clip(cols, 0, hi_ref[:, :30])

    ring = jnp.clip(cols[:, 5:6] + 1, 0, 1)

    c27 = cols[:, 27:28]
    m27 = jnp.full_like(c27, 6)
    for i, v in enumerate(_RINGSIZE_VALS):
        if i != 6:
            m27 = jnp.where(c27 == v, i, m27)

    idx_all = jnp.concatenate(
        [idx0, cl[:, 1:5], ring, cl[:, 6:27], m27, cl[:, 28:30],
         jnp.zeros((a.shape[0], 2), dtype=jnp.int32)], axis=1)
    idx_f = idx_all.astype(jnp.float32)

    bcast = jax.lax.dot(idx_f, sel_ref[...],
                        preferred_element_type=jnp.float32)
    onehot = (bcast == rvals_ref[...]).astype(jnp.float32)

    out = jax.lax.dot(onehot, wfull_ref[...],
                      preferred_element_type=jnp.float32)
    out = out + jax.lax.dot(a, wa_ref[...],
                            preferred_element_type=jnp.float32)
    o_ref[...] = out + bias_ref[...]


def kernel(atom_inputs, element_embed, degree_embed, ring_embed, charge_embed,
           aromatic_embed, hybrid_embed, hydrogen_embed, func_embeds,
           h_don_embed, h_acc_embed, ringsize_embed, aroma_num_embed,
           fused_if_embed, func_reduce_w, func_reduce_b, bond_env_w,
           bond_env_b):
    sel, rvals, w_full, w_a, bias = _build_constants(
        element_embed, degree_embed, ring_embed, charge_embed, aromatic_embed,
        hybrid_embed, hydrogen_embed, func_embeds, h_don_embed, h_acc_embed,
        ringsize_embed, aroma_num_embed, fused_if_embed, func_reduce_w,
        func_reduce_b, bond_env_w, bond_env_b)

    hi = jnp.array([[0, 6, 7, 5, 1, 0, 4] + [1] * 18 + [1, 1, 0, 4, 7, 0, 0]],
                   dtype=jnp.int32)

    n = atom_inputs.shape[0]
    assert n % _BLOCK == 0
    grid = (n // _BLOCK,)

    full = lambda shape: pl.BlockSpec(shape, lambda i: (0, 0))
    return pl.pallas_call(
        _body,
        grid=grid,
        in_specs=[
            pl.BlockSpec((_BLOCK, _IN_W), lambda i: (i, 0)),
            full((1, _NIDX)),
            full((_NIDX, _NPOS)),
            full((1, _NPOS)),
            full((_NPOS, _OUT_W)),
            full((_IN_W, _OUT_W)),
            full((1, _OUT_W)),
        ],
        out_specs=pl.BlockSpec((_BLOCK, _OUT_W), lambda i: (i, 0)),
        out_shape=jax.ShapeDtypeStruct((n, _OUT_W), jnp.float32),
    )(atom_inputs, hi, sel, rvals, w_full, w_a, bias)
